# baseline (speedup 1.0000x reference)
import functools

import jax
import jax.numpy as jnp
import numpy as np
from jax import lax
from jax.experimental import pallas as pl
from jax.experimental.pallas import tpu as pltpu
from jax.experimental.pallas import tpu_sc as plsc

_ANCHORS = np.array([[10.0, 13.0], [16.0, 30.0], [33.0, 23.0]], dtype=np.float32)
_IMG_DIM = 608.0
_NA = 3
_NW = 32
_X0S = (0, 16, 32, 48, 60)


def kernel(x):
    nB, C, nG, _ = x.shape
    attrs = C // _NA
    S = nG * nG
    stride = _IMG_DIM / nG
    RPS = 4
    n_strips = nG // RPS
    SPW = RPS * nG
    n_tasks = nB * _NA * n_strips
    n_per = -(-n_tasks // _NW)

    mesh = plsc.VectorSubcoreMesh(core_axis_name="c", subcore_axis_name="s")

    @functools.partial(
        pl.kernel,
        out_type=jax.ShapeDtypeStruct((nB, _NA * S, attrs), jnp.float32),
        mesh=mesh,
        scratch_types=[
            pltpu.VMEM((attrs, RPS, nG), jnp.float32),
            pltpu.VMEM((SPW, attrs), jnp.float32),
        ],
        compiler_params=pltpu.CompilerParams(use_tc_tiling_on_sc=True, needs_layout_passes=False),
    )
    def sc_k(x_hbm, out_hbm, in_v, out_v):
        wid = lax.axis_index("s") * 2 + lax.axis_index("c")
        iota = lax.iota(jnp.int32, 16)
        iotaf = iota.astype(jnp.float32)

        def task_body(i, carry):
            t = wid + _NW * i

            @pl.when(t < n_tasks)
            def _():
                b = t // (_NA * n_strips)
                r1 = t % (_NA * n_strips)
                a = r1 // n_strips
                st = r1 % n_strips
                y0 = st * RPS
                c0 = a * attrs
                row0 = a * S + st * SPW

                pltpu.sync_copy(x_hbm.at[b, pl.ds(c0, attrs), pl.ds(y0, RPS), :], in_v)

                aw = jnp.where(
                    a == 0, _ANCHORS[0, 0], jnp.where(a == 1, _ANCHORS[1, 0], _ANCHORS[2, 0])
                )
                ah = jnp.where(
                    a == 0, _ANCHORS[0, 1], jnp.where(a == 1, _ANCHORS[1, 1], _ANCHORS[2, 1])
                )
                y0f = y0.astype(jnp.float32)

                nchunk = len(_X0S)
                cpr = RPS * nchunk

                @plsc.parallel_loop(0, 4 * cpr, step=1, unroll=2)
                def box_body(i):
                    c = i // cpr
                    j = i % cpr
                    r = j // nchunk
                    xi = j % nchunk
                    x0 = jnp.where(xi == nchunk - 1, nG - 16, xi * 16)
                    v = in_v[c, r, pl.ds(x0, 16)]
                    em = jnp.exp(-v)
                    sig = 1.0 / (1.0 + em)
                    ex = 1.0 / em
                    gxv = x0.astype(jnp.float32) + iotaf
                    gyv = y0f + r.astype(jnp.float32)
                    res = jnp.where(
                        c == 0,
                        (sig + gxv) * stride,
                        jnp.where(
                            c == 1, (sig + gyv) * stride, jnp.where(c == 2, ex * aw, ex * ah)
                        ),
                    )
                    rows = (r * nG + x0) + iota
                    cols = jnp.full((16,), 0, dtype=jnp.int32) + c
                    plsc.store_scatter(out_v, [rows, cols], res)

                @plsc.parallel_loop(0, (attrs - 4) * cpr, step=1, unroll=4)
                def cls_body(i):
                    c = 4 + i // cpr
                    j = i % cpr
                    r = j // nchunk
                    xi = j % nchunk
                    x0 = jnp.where(xi == nchunk - 1, nG - 16, xi * 16)
                    v = in_v[c, r, pl.ds(x0, 16)]
                    res = 1.0 / (1.0 + jnp.exp(-v))
                    rows = (r * nG + x0) + iota
                    cols = jnp.full((16,), 0, dtype=jnp.int32) + c
                    plsc.store_scatter(out_v, [rows, cols], res)

                pltpu.sync_copy(out_v, out_hbm.at[b, pl.ds(row0, SPW), :])

            return carry

        lax.fori_loop(0, n_per, task_body, 0)

    return sc_k(x)


# speedup vs baseline: 1.1224x; 1.1224x over previous
import functools

import jax
import jax.numpy as jnp
import numpy as np
from jax import lax
from jax.experimental import pallas as pl
from jax.experimental.pallas import tpu as pltpu
from jax.experimental.pallas import tpu_sc as plsc

_ANCHORS = np.array([[10.0, 13.0], [16.0, 30.0], [33.0, 23.0]], dtype=np.float32)
_IMG_DIM = 608.0
_NA = 3
_NW = 32
_X0S = (0, 16, 32, 48, 60)


def kernel(x):
    nB, C, nG, _ = x.shape
    attrs = C // _NA
    S = nG * nG
    stride = _IMG_DIM / nG
    RPS = 4
    n_strips = nG // RPS
    SPW = RPS * nG
    n_tasks = nB * _NA * n_strips
    n_per = -(-n_tasks // _NW)

    mesh = plsc.VectorSubcoreMesh(core_axis_name="c", subcore_axis_name="s")

    @functools.partial(
        pl.kernel,
        out_type=jax.ShapeDtypeStruct((nB, _NA * S, attrs), jnp.float32),
        mesh=mesh,
        scratch_types=[
            pltpu.VMEM((attrs, RPS, nG), jnp.float32),
            pltpu.VMEM((SPW, attrs), jnp.float32),
        ],
        compiler_params=pltpu.CompilerParams(use_tc_tiling_on_sc=True, needs_layout_passes=False),
    )
    def sc_k(x_hbm, out_hbm, in_v, out_v):
        wid = lax.axis_index("s") * 2 + lax.axis_index("c")
        iota = lax.iota(jnp.int32, 16)
        iotaf = iota.astype(jnp.float32)

        def task_body(i, carry):
            t = wid + _NW * i

            @pl.when(t < n_tasks)
            def _():
                b = t // (_NA * n_strips)
                r1 = t % (_NA * n_strips)
                a = r1 // n_strips
                st = r1 % n_strips
                y0 = st * RPS
                c0 = a * attrs
                row0 = a * S + st * SPW

                pltpu.sync_copy(x_hbm.at[b, pl.ds(c0, attrs), pl.ds(y0, RPS), :], in_v)

                aw = jnp.where(
                    a == 0, _ANCHORS[0, 0], jnp.where(a == 1, _ANCHORS[1, 0], _ANCHORS[2, 0])
                )
                ah = jnp.where(
                    a == 0, _ANCHORS[0, 1], jnp.where(a == 1, _ANCHORS[1, 1], _ANCHORS[2, 1])
                )
                y0f = y0.astype(jnp.float32)

                nchunk = len(_X0S)
                cpr = RPS * nchunk

                @plsc.parallel_loop(0, 4 * cpr, step=1, unroll=2)
                def box_body(i):
                    c = i // cpr
                    j = i % cpr
                    r = j // nchunk
                    xi = j % nchunk
                    x0 = jnp.where(xi == nchunk - 1, nG - 16, xi * 16)
                    v = in_v[c, r, pl.ds(x0, 16)]
                    em = jnp.exp(-v)
                    sig = 1.0 / (1.0 + em)
                    ex = 1.0 / em
                    gxv = x0.astype(jnp.float32) + iotaf
                    gyv = y0f + r.astype(jnp.float32)
                    res = jnp.where(
                        c == 0,
                        (sig + gxv) * stride,
                        jnp.where(
                            c == 1, (sig + gyv) * stride, jnp.where(c == 2, ex * aw, ex * ah)
                        ),
                    )
                    rows = (r * nG + x0) + iota
                    cols = jnp.full((16,), 0, dtype=jnp.int32) + c
                    plsc.store_scatter(out_v, [rows, cols], res)

                @plsc.parallel_loop(4, attrs, step=1, unroll=4)
                def cls_body(c):
                    for r in range(RPS):
                        for x0 in _X0S:
                            v = in_v[c, r, pl.ds(x0, 16)]
                            res = 1.0 / (1.0 + jnp.exp(-v))
                            rows = (r * nG + x0) + iota
                            cols = jnp.full((16,), 0, dtype=jnp.int32) + c
                            plsc.store_scatter(out_v, [rows, cols], res)

                pltpu.sync_copy(out_v, out_hbm.at[b, pl.ds(row0, SPW), :])

            return carry

        lax.fori_loop(0, n_per, task_body, 0)

    return sc_k(x)


# speedup vs baseline: 1.8342x; 1.6341x over previous
import functools

import jax
import jax.numpy as jnp
import numpy as np
from jax import lax
from jax.experimental import pallas as pl
from jax.experimental.pallas import tpu as pltpu
from jax.experimental.pallas import tpu_sc as plsc

_ANCHORS = np.array([[10.0, 13.0], [16.0, 30.0], [33.0, 23.0]], dtype=np.float32)
_NUM_CLASSES = 80
_IMG_DIM = 608.0
_NA = 3
_NW = 32
_X0S = (0, 16, 32, 48, 60)
_TC_BATCHES = 11


def _yolo_tc_body(x_ref, o_ref, *, nG, stride):
    v = x_ref[0]
    nc = v.shape[0]
    attrs = nc // _NA

    rows = jax.lax.broadcasted_iota(jnp.int32, (nc, 1, 1), 0)
    r = rows % attrs
    gy = jax.lax.broadcasted_iota(jnp.int32, (1, nG, 1), 1).astype(jnp.float32)
    gx = jax.lax.broadcasted_iota(jnp.int32, (1, 1, nG), 2).astype(jnp.float32)

    sig = jax.nn.sigmoid(v)
    ex = jnp.exp(v)

    aw = jnp.where(rows < attrs, _ANCHORS[0, 0], jnp.where(rows < 2 * attrs, _ANCHORS[1, 0], _ANCHORS[2, 0]))
    ah = jnp.where(rows < attrs, _ANCHORS[0, 1], jnp.where(rows < 2 * attrs, _ANCHORS[1, 1], _ANCHORS[2, 1]))

    val = jnp.where(
        r == 0,
        (sig + gx) * stride,
        jnp.where(
            r == 1,
            (sig + gy) * stride,
            jnp.where(r == 2, ex * aw, jnp.where(r == 3, ex * ah, sig)),
        ),
    )
    S = nG * nG
    w = val.reshape(_NA, attrs, S)
    o_ref[0] = jnp.swapaxes(w, 1, 2).reshape(_NA * S, attrs)


def _tc_kernel(x):
    nB, C, nG, _ = x.shape
    attrs = C // _NA
    S = nG * nG
    stride = _IMG_DIM / nG

    return pl.pallas_call(
        functools.partial(_yolo_tc_body, nG=nG, stride=stride),
        grid=(nB,),
        in_specs=[pl.BlockSpec((1, C, nG, nG), lambda b: (b, 0, 0, 0))],
        out_specs=pl.BlockSpec((1, _NA * S, attrs), lambda b: (b, 0, 0)),
        out_shape=jax.ShapeDtypeStruct((nB, _NA * S, attrs), jnp.float32),
        compiler_params=pltpu.CompilerParams(dimension_semantics=("parallel",)),
    )(x)


def _sc_kernel(x):
    nB, C, nG, _ = x.shape
    attrs = C // _NA
    S = nG * nG
    stride = _IMG_DIM / nG
    RPS = 4
    n_strips = nG // RPS
    SPW = RPS * nG
    n_tasks = nB * _NA * n_strips
    n_per = -(-n_tasks // _NW)

    mesh = plsc.VectorSubcoreMesh(core_axis_name="c", subcore_axis_name="s")

    @functools.partial(
        pl.kernel,
        out_type=jax.ShapeDtypeStruct((nB, _NA * S, attrs), jnp.float32),
        mesh=mesh,
        scratch_types=[
            pltpu.VMEM((attrs, RPS, nG), jnp.float32),
            pltpu.VMEM((SPW, attrs), jnp.float32),
        ],
        compiler_params=pltpu.CompilerParams(use_tc_tiling_on_sc=True, needs_layout_passes=False),
    )
    def sc_k(x_hbm, out_hbm, in_v, out_v):
        wid = lax.axis_index("s") * 2 + lax.axis_index("c")
        iota = lax.iota(jnp.int32, 16)
        iotaf = iota.astype(jnp.float32)

        def task_body(i, carry):
            t = wid + _NW * i

            @pl.when(t < n_tasks)
            def _():
                b = t // (_NA * n_strips)
                r1 = t % (_NA * n_strips)
                a = r1 // n_strips
                st = r1 % n_strips
                y0 = st * RPS
                c0 = a * attrs
                row0 = a * S + st * SPW

                pltpu.sync_copy(x_hbm.at[b, pl.ds(c0, attrs), pl.ds(y0, RPS), :], in_v)

                aw = jnp.where(
                    a == 0, _ANCHORS[0, 0], jnp.where(a == 1, _ANCHORS[1, 0], _ANCHORS[2, 0])
                )
                ah = jnp.where(
                    a == 0, _ANCHORS[0, 1], jnp.where(a == 1, _ANCHORS[1, 1], _ANCHORS[2, 1])
                )
                y0f = y0.astype(jnp.float32)

                nchunk = len(_X0S)
                cpr = RPS * nchunk

                @plsc.parallel_loop(0, 4 * cpr, step=1, unroll=2)
                def box_body(i2):
                    c = i2 // cpr
                    j = i2 % cpr
                    r = j // nchunk
                    xi = j % nchunk
                    x0 = jnp.where(xi == nchunk - 1, nG - 16, xi * 16)
                    v = in_v[c, r, pl.ds(x0, 16)]
                    em = jnp.exp(-v)
                    sig = 1.0 / (1.0 + em)
                    ex = 1.0 / em
                    gxv = x0.astype(jnp.float32) + iotaf
                    gyv = y0f + r.astype(jnp.float32)
                    res = jnp.where(
                        c == 0,
                        (sig + gxv) * stride,
                        jnp.where(
                            c == 1, (sig + gyv) * stride, jnp.where(c == 2, ex * aw, ex * ah)
                        ),
                    )
                    rows = (r * nG + x0) + iota
                    cols = jnp.full((16,), 0, dtype=jnp.int32) + c
                    plsc.store_scatter(out_v, [rows, cols], res)

                @plsc.parallel_loop(4, attrs, step=1, unroll=4)
                def cls_body(c):
                    for r in range(RPS):
                        for x0 in _X0S:
                            v = in_v[c, r, pl.ds(x0, 16)]
                            res = 1.0 / (1.0 + jnp.exp(-v))
                            rows = (r * nG + x0) + iota
                            cols = jnp.full((16,), 0, dtype=jnp.int32) + c
                            plsc.store_scatter(out_v, [rows, cols], res)

                pltpu.sync_copy(out_v, out_hbm.at[b, pl.ds(row0, SPW), :])

            return carry

        lax.fori_loop(0, n_per, task_body, 0)

    return sc_k(x)


def kernel(x):
    k = _TC_BATCHES
    out_tc = _tc_kernel(x[:k])
    out_sc = _sc_kernel(x[k:])
    return jnp.concatenate([out_tc, out_sc], axis=0)


# speedup vs baseline: 2.1089x; 1.1497x over previous
import functools

import jax
import jax.numpy as jnp
import numpy as np
from jax import lax
from jax.experimental import pallas as pl
from jax.experimental.pallas import tpu as pltpu
from jax.experimental.pallas import tpu_sc as plsc

_ANCHORS = np.array([[10.0, 13.0], [16.0, 30.0], [33.0, 23.0]], dtype=np.float32)
_NUM_CLASSES = 80
_IMG_DIM = 608.0
_NA = 3
_NW = 32
_X0S = (0, 16, 32, 48, 60)
_TC_BATCHES = 11


def _yolo_tc_body(x_ref, o_ref, *, nG, stride):
    v = x_ref[0]
    nc = v.shape[0]
    attrs = nc // _NA

    rows = jax.lax.broadcasted_iota(jnp.int32, (nc, 1, 1), 0)
    r = rows % attrs
    gy = jax.lax.broadcasted_iota(jnp.int32, (1, nG, 1), 1).astype(jnp.float32)
    gx = jax.lax.broadcasted_iota(jnp.int32, (1, 1, nG), 2).astype(jnp.float32)

    sig = jax.nn.sigmoid(v)
    ex = jnp.exp(v)

    aw = jnp.where(rows < attrs, _ANCHORS[0, 0], jnp.where(rows < 2 * attrs, _ANCHORS[1, 0], _ANCHORS[2, 0]))
    ah = jnp.where(rows < attrs, _ANCHORS[0, 1], jnp.where(rows < 2 * attrs, _ANCHORS[1, 1], _ANCHORS[2, 1]))

    val = jnp.where(
        r == 0,
        (sig + gx) * stride,
        jnp.where(
            r == 1,
            (sig + gy) * stride,
            jnp.where(r == 2, ex * aw, jnp.where(r == 3, ex * ah, sig)),
        ),
    )
    S = nG * nG
    w = val.reshape(_NA, attrs, S)
    o_ref[0] = jnp.swapaxes(w, 1, 2).reshape(_NA * S, attrs)


def _tc_kernel(x, out_batches):
    nB, C, nG, _ = x.shape
    attrs = C // _NA
    S = nG * nG
    stride = _IMG_DIM / nG

    return pl.pallas_call(
        functools.partial(_yolo_tc_body, nG=nG, stride=stride),
        grid=(nB,),
        in_specs=[pl.BlockSpec((1, C, nG, nG), lambda b: (b, 0, 0, 0))],
        out_specs=pl.BlockSpec((1, _NA * S, attrs), lambda b: (b, 0, 0)),
        out_shape=jax.ShapeDtypeStruct((out_batches, _NA * S, attrs), jnp.float32),
        compiler_params=pltpu.CompilerParams(dimension_semantics=("parallel",)),
    )(x)


def _sc_kernel(x):
    nB, C, nG, _ = x.shape
    attrs = C // _NA
    S = nG * nG
    stride = _IMG_DIM / nG
    RPS = 4
    n_strips = nG // RPS
    SPW = RPS * nG
    n_tasks = nB * _NA * n_strips
    n_per = -(-n_tasks // _NW)

    mesh = plsc.VectorSubcoreMesh(core_axis_name="c", subcore_axis_name="s")

    @functools.partial(
        pl.kernel,
        out_type=jax.ShapeDtypeStruct((nB, _NA * S, attrs), jnp.float32),
        mesh=mesh,
        scratch_types=[
            pltpu.VMEM((attrs, RPS, nG), jnp.float32),
            pltpu.VMEM((SPW, attrs), jnp.float32),
        ],
        compiler_params=pltpu.CompilerParams(use_tc_tiling_on_sc=True, needs_layout_passes=False),
    )
    def sc_k(x_hbm, out_hbm, in_v, out_v):
        wid = lax.axis_index("s") * 2 + lax.axis_index("c")
        iota = lax.iota(jnp.int32, 16)
        iotaf = iota.astype(jnp.float32)

        def task_body(i, carry):
            t = wid + _NW * i

            @pl.when(t < n_tasks)
            def _():
                b = t // (_NA * n_strips)
                r1 = t % (_NA * n_strips)
                a = r1 // n_strips
                st = r1 % n_strips
                y0 = st * RPS
                c0 = a * attrs
                row0 = a * S + st * SPW

                pltpu.sync_copy(x_hbm.at[b, pl.ds(c0, attrs), pl.ds(y0, RPS), :], in_v)

                aw = jnp.where(
                    a == 0, _ANCHORS[0, 0], jnp.where(a == 1, _ANCHORS[1, 0], _ANCHORS[2, 0])
                )
                ah = jnp.where(
                    a == 0, _ANCHORS[0, 1], jnp.where(a == 1, _ANCHORS[1, 1], _ANCHORS[2, 1])
                )
                y0f = y0.astype(jnp.float32)

                nchunk = len(_X0S)
                cpr = RPS * nchunk

                @plsc.parallel_loop(0, 4 * cpr, step=1, unroll=2)
                def box_body(i2):
                    c = i2 // cpr
                    j = i2 % cpr
                    r = j // nchunk
                    xi = j % nchunk
                    x0 = jnp.where(xi == nchunk - 1, nG - 16, xi * 16)
                    v = in_v[c, r, pl.ds(x0, 16)]
                    em = jnp.exp(-v)
                    sig = 1.0 / (1.0 + em)
                    ex = 1.0 / em
                    gxv = x0.astype(jnp.float32) + iotaf
                    gyv = y0f + r.astype(jnp.float32)
                    res = jnp.where(
                        c == 0,
                        (sig + gxv) * stride,
                        jnp.where(
                            c == 1, (sig + gyv) * stride, jnp.where(c == 2, ex * aw, ex * ah)
                        ),
                    )
                    rows = (r * nG + x0) + iota
                    cols = jnp.full((16,), 0, dtype=jnp.int32) + c
                    plsc.store_scatter(out_v, [rows, cols], res)

                @plsc.parallel_loop(4, attrs, step=1, unroll=4)
                def cls_body(c):
                    for r in range(RPS):
                        for x0 in _X0S:
                            v = in_v[c, r, pl.ds(x0, 16)]
                            res = 1.0 / (1.0 + jnp.exp(-v))
                            rows = (r * nG + x0) + iota
                            cols = jnp.full((16,), 0, dtype=jnp.int32) + c
                            plsc.store_scatter(out_v, [rows, cols], res)

                pltpu.sync_copy(out_v, out_hbm.at[b, pl.ds(row0, SPW), :])

            return carry

        lax.fori_loop(0, n_per, task_body, 0)

    return sc_k(x)


def kernel(x):
    k = _TC_BATCHES
    nB = x.shape[0]
    out_full = _tc_kernel(x[:k], nB)
    out_sc = _sc_kernel(x[k:])
    return lax.dynamic_update_slice(out_full, out_sc, (k, 0, 0))


# speedup vs baseline: 2.5199x; 1.1949x over previous
import functools

import jax
import jax.numpy as jnp
import numpy as np
from jax.experimental import pallas as pl
from jax.experimental.pallas import tpu as pltpu

_ANCHORS = np.array([[10.0, 13.0], [16.0, 30.0], [33.0, 23.0]], dtype=np.float32)
_NUM_CLASSES = 80
_IMG_DIM = 608.0
_NA = 3


def _yolo_body(x_ref, o_ref, *, nG, stride):
    v = x_ref[0]
    nc = v.shape[0]
    attrs = nc // _NA

    gy = jax.lax.broadcasted_iota(jnp.int32, (1, nG, 1), 1).astype(jnp.float32)
    gx = jax.lax.broadcasted_iota(jnp.int32, (1, 1, nG), 2).astype(jnp.float32)

    sig = jax.nn.sigmoid(v)

    parts = []
    for a in range(_NA):
        base = a * attrs
        vx = v[base + 0]
        vy = v[base + 1]
        bx = (jax.nn.sigmoid(vx) + gx[0]) * stride
        by = (jax.nn.sigmoid(vy) + gy[0]) * stride
        bw = jnp.exp(v[base + 2]) * _ANCHORS[a, 0]
        bh = jnp.exp(v[base + 3]) * _ANCHORS[a, 1]
        parts.append(jnp.stack([bx, by, bw, bh], axis=0))
        parts.append(sig[base + 4 : base + attrs])
    val = jnp.concatenate(parts, axis=0)

    S = nG * nG
    w = val.reshape(_NA, attrs, S)
    o_ref[0] = jnp.swapaxes(w, 1, 2).reshape(_NA * S, attrs)


def kernel(x):
    nB, C, nG, _ = x.shape
    nA = _NA
    attrs = C // nA
    S = nG * nG
    stride = _IMG_DIM / nG

    return pl.pallas_call(
        functools.partial(_yolo_body, nG=nG, stride=stride),
        grid=(nB,),
        in_specs=[pl.BlockSpec((1, C, nG, nG), lambda b: (b, 0, 0, 0))],
        out_specs=pl.BlockSpec((1, nA * S, attrs), lambda b: (b, 0, 0)),
        out_shape=jax.ShapeDtypeStruct((nB, nA * S, attrs), jnp.float32),
        compiler_params=pltpu.CompilerParams(dimension_semantics=("parallel",)),
    )(x)
